# Initial kernel scaffold; baseline (speedup 1.0000x reference)
import jax
import jax.numpy as jnp
from jax.experimental import pallas as pl


def kernel(source, target, score, node_embedding, node_noise_factors, feature_noise_factors):
    raise NotImplementedError("write your pallas kernel here")


# speedup vs baseline: 1.1497x; 1.1497x over previous
import functools

import jax
import jax.numpy as jnp
from jax import lax
from jax.experimental import pallas as pl
from jax.experimental.pallas import tpu as pltpu
from jax.experimental.pallas import tpu_sc as plsc

K = 4096
DIM = 128
NC = 2
NS = 16
NW = NC * NS
KPW = K // NW
LANES = 16
CHUNKS = DIM // LANES


def _sc_dots_body(tgt_hbm, flag_hbm, srcidx_hbm, nnf_hbm, fnf_hbm, emb_hbm,
                  out_hbm, idx_v, rows_v, flag_v, srcidx_v, srcrow_v, tbuf_v,
                  dots_v, sem):
    wid = lax.axis_index("s") * NC + lax.axis_index("c")
    base = wid * KPW

    pltpu.sync_copy(tgt_hbm.at[pl.ds(base, KPW)], idx_v)
    pltpu.sync_copy(flag_hbm, flag_v)
    pltpu.sync_copy(srcidx_hbm, srcidx_v)

    flag = flag_v[pl.ds(0, LANES)][0]

    @pl.when(flag != 0)
    def _():
        pltpu.async_copy(nnf_hbm.at[idx_v], rows_v, sem).wait()

    @pl.when(flag == 0)
    def _():
        pltpu.async_copy(fnf_hbm.at[idx_v], rows_v, sem).wait()

    pltpu.async_copy(emb_hbm.at[srcidx_v], srcrow_v, sem).wait()

    s_chunks = [srcrow_v[0, pl.ds(c * LANES, LANES)] for c in range(CHUNKS)]
    lane_ids = lax.iota(jnp.int32, LANES)

    def group_body(g, carry):
        for i in range(LANES):
            r = g * LANES + i
            p = rows_v[r, pl.ds(0, LANES)] * s_chunks[0]
            for c in range(1, CHUNKS):
                p = p + rows_v[r, pl.ds(c * LANES, LANES)] * s_chunks[c]
            plsc.store_scatter(tbuf_v, [lane_ids, jnp.full((LANES,), i, jnp.int32)], p)
        acc = tbuf_v[0, pl.ds(0, LANES)]
        for i in range(1, LANES):
            acc = acc + tbuf_v[i, pl.ds(0, LANES)]
        dots_v[pl.ds(g * LANES, LANES)] = acc
        return carry

    lax.fori_loop(0, KPW // LANES, group_body, 0)

    pltpu.sync_copy(dots_v, out_hbm.at[pl.ds(base, KPW)])


_sc_dots = pl.kernel(
    _sc_dots_body,
    out_type=jax.ShapeDtypeStruct((K,), jnp.float32),
    mesh=plsc.VectorSubcoreMesh(core_axis_name="c", subcore_axis_name="s"),
    scratch_types=[
        pltpu.VMEM((KPW,), jnp.int32),
        pltpu.VMEM((KPW, DIM), jnp.float32),
        pltpu.VMEM((LANES,), jnp.int32),
        pltpu.VMEM((1,), jnp.int32),
        pltpu.VMEM((1, DIM), jnp.float32),
        pltpu.VMEM((LANES, LANES), jnp.float32),
        pltpu.VMEM((KPW,), jnp.float32),
        pltpu.SemaphoreType.DMA,
    ],
    compiler_params=pltpu.CompilerParams(needs_layout_passes=False),
)


def _tc_loss_body(dots_ref, o_ref):
    x = dots_ref[:]
    c = jnp.clip(x, -20.0, 20.0)
    s = jax.nn.sigmoid(c)
    row = lax.broadcasted_iota(jnp.int32, (NW, KPW), 0)
    col = lax.broadcasted_iota(jnp.int32, (NW, KPW), 1)
    first = (row == 0) & (col == 0)
    term = jnp.where(first, jnp.log(s), jnp.log(1.0 - s))
    o_ref[0, 0] = -jnp.sum(term) / float(K)


_tc_loss = pl.pallas_call(
    _tc_loss_body,
    out_shape=jax.ShapeDtypeStruct((1, 1), jnp.float32),
    out_specs=pl.BlockSpec(memory_space=pltpu.SMEM),
)


def kernel(source, target, score, node_embedding, node_noise_factors,
           feature_noise_factors):
    tgt = target.astype(jnp.int32)
    srcidx = source.astype(jnp.int32)
    flag8 = jnp.broadcast_to(
        (jnp.asarray(score) > 0.5).astype(jnp.int32), (LANES,))
    dots = _sc_dots(tgt, flag8, srcidx, node_noise_factors,
                    feature_noise_factors, node_embedding)
    loss = _tc_loss(dots.reshape(NW, KPW))
    return loss[0, 0]
